# baseline (speedup 1.0000x reference)
import jax
import jax.numpy as jnp
from jax.experimental import pallas as pl

_BM = 512


def _mlp_block(x_ref, m_ref, w_ref, w1_ref, b1_ref, w2_ref, b2_ref, w3_ref,
               b3_ref, o_ref):
    bf = jnp.bfloat16
    xm = (x_ref[:] * m_ref[:]).astype(bf)
    h = jnp.dot(xm, w_ref[:].astype(bf), preferred_element_type=jnp.float32)
    h = jnp.maximum(
        jnp.dot(h.astype(bf), w1_ref[:].astype(bf),
                preferred_element_type=jnp.float32) + b1_ref[:], 0.0)
    h = jnp.maximum(
        jnp.dot(h.astype(bf), w2_ref[:].astype(bf),
                preferred_element_type=jnp.float32) + b2_ref[:], 0.0)
    o_ref[:] = (jnp.dot(h.astype(bf), w3_ref[:].astype(bf),
                        preferred_element_type=jnp.float32) + b3_ref[:])


def kernel(x, feature_mask, W, W1, b1, W2, b2, W3, b3):
    batch, feat = x.shape
    hidden = W.shape[1]
    classes = W3.shape[1]
    mask_f = feature_mask.astype(jnp.float32).reshape(1, feat)
    b1r = b1.reshape(1, hidden)
    b2r = b2.reshape(1, hidden)
    b3r = b3.reshape(1, classes)
    bm = min(_BM, batch)
    grid = (batch // bm,)
    return pl.pallas_call(
        _mlp_block,
        grid=grid,
        in_specs=[
            pl.BlockSpec((bm, feat), lambda i: (i, 0)),
            pl.BlockSpec((1, feat), lambda i: (0, 0)),
            pl.BlockSpec((feat, hidden), lambda i: (0, 0)),
            pl.BlockSpec((hidden, hidden), lambda i: (0, 0)),
            pl.BlockSpec((1, hidden), lambda i: (0, 0)),
            pl.BlockSpec((hidden, hidden), lambda i: (0, 0)),
            pl.BlockSpec((1, hidden), lambda i: (0, 0)),
            pl.BlockSpec((hidden, classes), lambda i: (0, 0)),
            pl.BlockSpec((1, classes), lambda i: (0, 0)),
        ],
        out_specs=pl.BlockSpec((bm, classes), lambda i: (i, 0)),
        out_shape=jax.ShapeDtypeStruct((batch, classes), x.dtype),
    )(x, mask_f, W, W1, b1r, W2, b2r, W3, b3r)


# speedup vs baseline: 1.1525x; 1.1525x over previous
import jax
import jax.numpy as jnp
from jax.experimental import pallas as pl

_BM = 1024


def _mlp_block(x_ref, m_ref, w_ref, w1_ref, b1_ref, w2_ref, b2_ref, w3_ref,
               b3_ref, o_ref):
    bf = jnp.bfloat16
    xm = (x_ref[:] * m_ref[:]).astype(bf)
    h = jnp.dot(xm, w_ref[:].astype(bf), preferred_element_type=jnp.float32)
    h = jnp.maximum(
        jnp.dot(h.astype(bf), w1_ref[:].astype(bf),
                preferred_element_type=jnp.float32) + b1_ref[:], 0.0)
    h = jnp.maximum(
        jnp.dot(h.astype(bf), w2_ref[:].astype(bf),
                preferred_element_type=jnp.float32) + b2_ref[:], 0.0)
    o_ref[:] = (jnp.dot(h.astype(bf), w3_ref[:].astype(bf),
                        preferred_element_type=jnp.float32) + b3_ref[:])


def kernel(x, feature_mask, W, W1, b1, W2, b2, W3, b3):
    batch, feat = x.shape
    hidden = W.shape[1]
    classes = W3.shape[1]
    mask_f = feature_mask.astype(jnp.float32).reshape(1, feat)
    b1r = b1.reshape(1, hidden)
    b2r = b2.reshape(1, hidden)
    b3r = b3.reshape(1, classes)
    bm = min(_BM, batch)
    grid = (batch // bm,)
    return pl.pallas_call(
        _mlp_block,
        grid=grid,
        in_specs=[
            pl.BlockSpec((bm, feat), lambda i: (i, 0)),
            pl.BlockSpec((1, feat), lambda i: (0, 0)),
            pl.BlockSpec((feat, hidden), lambda i: (0, 0)),
            pl.BlockSpec((hidden, hidden), lambda i: (0, 0)),
            pl.BlockSpec((1, hidden), lambda i: (0, 0)),
            pl.BlockSpec((hidden, hidden), lambda i: (0, 0)),
            pl.BlockSpec((1, hidden), lambda i: (0, 0)),
            pl.BlockSpec((hidden, classes), lambda i: (0, 0)),
            pl.BlockSpec((1, classes), lambda i: (0, 0)),
        ],
        out_specs=pl.BlockSpec((bm, classes), lambda i: (i, 0)),
        out_shape=jax.ShapeDtypeStruct((batch, classes), x.dtype),
    )(x, mask_f, W, W1, b1r, W2, b2r, W3, b3r)
